# Initial kernel scaffold; baseline (speedup 1.0000x reference)
import jax
import jax.numpy as jnp
from jax.experimental import pallas as pl


def kernel(node_positions, adjacency, edge_index, weight):
    raise NotImplementedError("write your pallas kernel here")


# speedup vs baseline: 2.1861x; 2.1861x over previous
import jax
import jax.numpy as jnp
from jax import lax
from jax.experimental import pallas as pl
from jax.experimental.pallas import tpu as pltpu

EPS = 0.001
PROX = 0.15

E = 2048
TR = 256
PT = 256
NF = 16


def _prep_body(pos_ref, srcf_ref, dstf_ref, rows_ref, cols_ref):
    pos = pos_ref[...]
    sf = srcf_ref[...]
    df = dstf_ref[...]
    node_ids = lax.broadcasted_iota(jnp.int32, (PT, E), 1).astype(jnp.float32)
    oh_s = (sf == node_ids).astype(jnp.float32)
    oh_d = (df == node_ids).astype(jnp.float32)
    dn = (((1,), (0,)), ((), ()))
    ps = lax.dot_general(oh_s, pos, dn, preferred_element_type=jnp.float32)
    pd = lax.dot_general(oh_d, pos, dn, preferred_element_type=jnp.float32)
    sx = ps[:, 0:1]
    sy = ps[:, 1:2]
    ex = pd[:, 0:1]
    ey = pd[:, 1:2]
    dx = ex - sx
    dy = ey - sy
    len2 = dx * dx + dy * dy
    a = jnp.maximum(len2, 1e-12)
    ra = 1.0 / a
    mx = (sx + ex) * 0.5
    my = (sy + ey) * 0.5
    h = jnp.sqrt(len2) * 0.5
    z = jnp.zeros_like(sx)
    tile = jnp.concatenate(
        [sx, sy, dx, dy, a, ra, mx, my, h, sf, df, z, z, z, z, z], axis=1)
    rows_ref[...] = tile
    cols_ref[...] = tile.T


def _pair_body(rows_ref, cols_ref, out_ref, acc_ref):
    i = pl.program_id(0)

    @pl.when(i == 0)
    def _init():
        acc_ref[0] = 0.0
        acc_ref[1] = 0.0

    rb = rows_ref[...]
    cb = cols_ref[...]
    sxi = rb[:, 0:1]
    syi = rb[:, 1:2]
    dxi = rb[:, 2:3]
    dyi = rb[:, 3:4]
    ai = rb[:, 4:5]
    mxi = rb[:, 6:7]
    myi = rb[:, 7:8]
    hi = rb[:, 8:9]
    sfi = rb[:, 9:10]
    dfi = rb[:, 10:11]
    sxj = cb[0:1, :]
    syj = cb[1:2, :]
    dxj = cb[2:3, :]
    dyj = cb[3:4, :]
    ej = cb[4:5, :]
    mxj = cb[6:7, :]
    myj = cb[7:8, :]
    hj = cb[8:9, :]
    sfj = cb[9:10, :]
    dfj = cb[10:11, :]

    adj = ((sfi == sfj) | (sfi == dfj) | (dfi == sfj) | (dfi == dfj))
    row_ids = i * TR + lax.broadcasted_iota(jnp.int32, (TR, 1), 0)
    col_ids = lax.broadcasted_iota(jnp.int32, (1, E), 1)
    tri = col_ids > row_ids
    mdx = mxi - mxj
    mdy = myi - myj
    md2 = mdx * mdx + mdy * mdy
    prox = hi + hj + PROX
    mask = (~adj) & tri & (md2 < prox * prox)
    maskf = mask.astype(jnp.float32)

    b = dxi * dxj + dyi * dyj
    rx = sxi - sxj
    ry = syi - syj
    c = dxi * rx + dyi * ry
    f = dxj * rx + dyj * ry
    denom = jnp.maximum(ai * ej - b * b, 1e-12)
    s = jnp.clip((b * f - c * ej) / denom, 0.0, 1.0)
    t = jnp.clip((b * s + f) / ej, 0.0, 1.0)
    s = jnp.clip((b * t - c) / ai, 0.0, 1.0)
    ddx = (sxi + s * dxi) - (sxj + t * dxj)
    ddy = (syi + s * dyi) - (syj + t * dyj)
    sq = ddx * ddx + ddy * ddy
    dist = jnp.sqrt(jnp.maximum(sq, 1e-24))
    per = jnp.maximum(EPS - dist, 0.0) * maskf

    acc_ref[0] += jnp.sum(per)
    acc_ref[1] += jnp.sum(maskf)

    @pl.when(i == pl.num_programs(0) - 1)
    def _final():
        total = acc_ref[0]
        cnt = acc_ref[1]
        loss = jnp.where(cnt > 0.0, total / jnp.maximum(cnt, 1.0), 0.0)
        out_ref[...] = loss.reshape(1, 1)


def kernel(node_positions, adjacency, edge_index, weight):
    del adjacency, weight
    pos = node_positions.reshape(-1, 2)
    srcf = edge_index[0].astype(jnp.float32).reshape(E, 1)
    dstf = edge_index[1].astype(jnp.float32).reshape(E, 1)

    rows, cols = pl.pallas_call(
        _prep_body,
        grid=(E // PT,),
        in_specs=[
            pl.BlockSpec((E, 2), lambda i: (0, 0)),
            pl.BlockSpec((PT, 1), lambda i: (i, 0)),
            pl.BlockSpec((PT, 1), lambda i: (i, 0)),
        ],
        out_specs=[
            pl.BlockSpec((PT, NF), lambda i: (i, 0)),
            pl.BlockSpec((NF, PT), lambda i: (0, i)),
        ],
        out_shape=[
            jax.ShapeDtypeStruct((E, NF), jnp.float32),
            jax.ShapeDtypeStruct((NF, E), jnp.float32),
        ],
    )(pos, srcf, dstf)

    loss = pl.pallas_call(
        _pair_body,
        grid=(E // TR,),
        in_specs=[
            pl.BlockSpec((TR, NF), lambda i: (i, 0)),
            pl.BlockSpec((NF, E), lambda i: (0, 0)),
        ],
        out_specs=pl.BlockSpec((1, 1), lambda i: (0, 0)),
        out_shape=jax.ShapeDtypeStruct((1, 1), jnp.float32),
        scratch_shapes=[pltpu.SMEM((2,), jnp.float32)],
    )(rows, cols)
    return loss.reshape(())


# speedup vs baseline: 2.3386x; 1.0697x over previous
import jax
import jax.numpy as jnp
from jax import lax
from jax.experimental import pallas as pl
from jax.experimental.pallas import tpu as pltpu

EPS = 0.001
PROX = 0.15

E = 2048
TR = 256
TCOL = 256
PT = 256
NF = 16


def _prep_body(pos_ref, srcf_ref, dstf_ref, rows_ref, cols_ref):
    pos = pos_ref[...]
    sf = srcf_ref[...]
    df = dstf_ref[...]
    node_ids = lax.broadcasted_iota(jnp.int32, (PT, E), 1).astype(jnp.float32)
    oh_s = (sf == node_ids).astype(jnp.float32)
    oh_d = (df == node_ids).astype(jnp.float32)
    dn = (((1,), (0,)), ((), ()))
    ps = lax.dot_general(oh_s, pos, dn, preferred_element_type=jnp.float32)
    pd = lax.dot_general(oh_d, pos, dn, preferred_element_type=jnp.float32)
    sx = ps[:, 0:1]
    sy = ps[:, 1:2]
    ex = pd[:, 0:1]
    ey = pd[:, 1:2]
    dx = ex - sx
    dy = ey - sy
    len2 = dx * dx + dy * dy
    a = jnp.maximum(len2, 1e-12)
    ra = 1.0 / a
    mx = (sx + ex) * 0.5
    my = (sy + ey) * 0.5
    h = jnp.sqrt(len2) * 0.5
    z = jnp.zeros_like(sx)
    tile = jnp.concatenate(
        [sx, sy, dx, dy, a, ra, mx, my, h, sf, df, z, z, z, z, z], axis=1)
    rows_ref[...] = tile
    cols_ref[...] = tile.T


def _pair_body(rows_ref, cols_ref, out_ref, acc_ref):
    i = pl.program_id(0)
    j = pl.program_id(1)

    @pl.when((i == 0) & (j == 0))
    def _init():
        acc_ref[0] = 0.0
        acc_ref[1] = 0.0

    @pl.when(j >= i)
    def _compute():
        rb = rows_ref[...]
        cb = cols_ref[...]
        sxi = rb[:, 0:1]
        syi = rb[:, 1:2]
        dxi = rb[:, 2:3]
        dyi = rb[:, 3:4]
        ai = rb[:, 4:5]
        rai = rb[:, 5:6]
        mxi = rb[:, 6:7]
        myi = rb[:, 7:8]
        hi = rb[:, 8:9]
        sfi = rb[:, 9:10]
        dfi = rb[:, 10:11]
        sxj = cb[0:1, :]
        syj = cb[1:2, :]
        dxj = cb[2:3, :]
        dyj = cb[3:4, :]
        ej = cb[4:5, :]
        rej = cb[5:6, :]
        mxj = cb[6:7, :]
        myj = cb[7:8, :]
        hj = cb[8:9, :]
        sfj = cb[9:10, :]
        dfj = cb[10:11, :]

        adj = ((sfi == sfj) | (sfi == dfj) | (dfi == sfj) | (dfi == dfj))
        row_ids = i * TR + lax.broadcasted_iota(jnp.int32, (TR, 1), 0)
        col_ids = j * TCOL + lax.broadcasted_iota(jnp.int32, (1, TCOL), 1)
        tri = col_ids > row_ids
        mdx = mxi - mxj
        mdy = myi - myj
        md2 = mdx * mdx + mdy * mdy
        prox = hi + hj + PROX
        mask = (~adj) & tri & (md2 < prox * prox)
        maskf = mask.astype(jnp.float32)

        b = dxi * dxj + dyi * dyj
        rx = sxi - sxj
        ry = syi - syj
        c = dxi * rx + dyi * ry
        f = dxj * rx + dyj * ry
        rdenom = 1.0 / jnp.maximum(ai * ej - b * b, 1e-12)
        s = jnp.clip((b * f - c * ej) * rdenom, 0.0, 1.0)
        t = jnp.clip((b * s + f) * rej, 0.0, 1.0)
        s = jnp.clip((b * t - c) * rai, 0.0, 1.0)
        ddx = (sxi + s * dxi) - (sxj + t * dxj)
        ddy = (syi + s * dyi) - (syj + t * dyj)
        sq = ddx * ddx + ddy * ddy
        dist = jnp.sqrt(jnp.maximum(sq, 1e-24))
        per = jnp.maximum(EPS - dist, 0.0) * maskf

        acc_ref[0] += jnp.sum(per)
        acc_ref[1] += jnp.sum(maskf)

    @pl.when((i == pl.num_programs(0) - 1) & (j == pl.num_programs(1) - 1))
    def _final():
        total = acc_ref[0]
        cnt = acc_ref[1]
        loss = jnp.where(cnt > 0.0, total / jnp.maximum(cnt, 1.0), 0.0)
        out_ref[...] = loss.reshape(1, 1)


def kernel(node_positions, adjacency, edge_index, weight):
    del adjacency, weight
    pos = node_positions.reshape(-1, 2)
    srcf = edge_index[0].astype(jnp.float32).reshape(E, 1)
    dstf = edge_index[1].astype(jnp.float32).reshape(E, 1)

    rows, cols = pl.pallas_call(
        _prep_body,
        grid=(E // PT,),
        in_specs=[
            pl.BlockSpec((E, 2), lambda i: (0, 0)),
            pl.BlockSpec((PT, 1), lambda i: (i, 0)),
            pl.BlockSpec((PT, 1), lambda i: (i, 0)),
        ],
        out_specs=[
            pl.BlockSpec((PT, NF), lambda i: (i, 0)),
            pl.BlockSpec((NF, PT), lambda i: (0, i)),
        ],
        out_shape=[
            jax.ShapeDtypeStruct((E, NF), jnp.float32),
            jax.ShapeDtypeStruct((NF, E), jnp.float32),
        ],
    )(pos, srcf, dstf)

    loss = pl.pallas_call(
        _pair_body,
        grid=(E // TR, E // TCOL),
        in_specs=[
            pl.BlockSpec((TR, NF), lambda i, j: (i, 0)),
            pl.BlockSpec((NF, TCOL), lambda i, j: (0, j)),
        ],
        out_specs=pl.BlockSpec((1, 1), lambda i, j: (0, 0)),
        out_shape=jax.ShapeDtypeStruct((1, 1), jnp.float32),
        scratch_shapes=[pltpu.SMEM((2,), jnp.float32)],
    )(rows, cols)
    return loss.reshape(())


# speedup vs baseline: 2.6309x; 1.1250x over previous
import jax
import jax.numpy as jnp
from jax import lax
from jax.experimental import pallas as pl
from jax.experimental.pallas import tpu as pltpu

EPS = 0.001
PROX = 0.15

E = 2048
TR = 256
TCOL = 256
SUB = 32
PT = 256
NF = 16


def _prep_body(pos_ref, srcf_ref, dstf_ref, rows_ref, cols_ref):
    pos = pos_ref[...]
    sf = srcf_ref[...]
    df = dstf_ref[...]
    node_ids = lax.broadcasted_iota(jnp.int32, (PT, E), 1).astype(jnp.float32)
    oh_s = (sf == node_ids).astype(jnp.float32)
    oh_d = (df == node_ids).astype(jnp.float32)
    dn = (((1,), (0,)), ((), ()))
    ps = lax.dot_general(oh_s, pos, dn, preferred_element_type=jnp.float32)
    pd = lax.dot_general(oh_d, pos, dn, preferred_element_type=jnp.float32)
    sx = ps[:, 0:1]
    sy = ps[:, 1:2]
    ex = pd[:, 0:1]
    ey = pd[:, 1:2]
    dx = ex - sx
    dy = ey - sy
    len2 = dx * dx + dy * dy
    a = jnp.maximum(len2, 1e-12)
    ra = 1.0 / a
    mx = (sx + ex) * 0.5
    my = (sy + ey) * 0.5
    h = jnp.sqrt(len2) * 0.5
    z = jnp.zeros_like(sx)
    tile = jnp.concatenate(
        [sx, sy, dx, dy, a, ra, mx, my, h, sf, df, z, z, z, z, z], axis=1)
    rows_ref[...] = tile
    cols_ref[...] = tile.T


def _pair_body(rows_ref, cols_ref, out_ref, accs_ref, accc_ref):
    i = pl.program_id(0)
    j = pl.program_id(1)

    @pl.when((i == 0) & (j == 0))
    def _init():
        accs_ref[...] = jnp.zeros((SUB, TCOL), jnp.float32)
        accc_ref[...] = jnp.zeros((SUB, TCOL), jnp.float32)

    @pl.when(j >= i)
    def _compute():
        cb = cols_ref[...]
        sxj = cb[0:1, :]
        syj = cb[1:2, :]
        dxj = cb[2:3, :]
        dyj = cb[3:4, :]
        ej = cb[4:5, :]
        rej = cb[5:6, :]
        mxj = cb[6:7, :]
        myj = cb[7:8, :]
        hj = cb[8:9, :]
        sfj = cb[9:10, :]
        dfj = cb[10:11, :]
        col_ids = j * TCOL + lax.broadcasted_iota(jnp.int32, (1, TCOL), 1)

        tp = jnp.zeros((SUB, TCOL), jnp.float32)
        tc = jnp.zeros((SUB, TCOL), jnp.float32)
        for k in range(TR // SUB):
            rb = rows_ref[k * SUB:(k + 1) * SUB, :]
            sxi = rb[:, 0:1]
            syi = rb[:, 1:2]
            dxi = rb[:, 2:3]
            dyi = rb[:, 3:4]
            ai = rb[:, 4:5]
            rai = rb[:, 5:6]
            mxi = rb[:, 6:7]
            myi = rb[:, 7:8]
            hi = rb[:, 8:9]
            sfi = rb[:, 9:10]
            dfi = rb[:, 10:11]

            adj = ((sfi == sfj) | (sfi == dfj) | (dfi == sfj) | (dfi == dfj))
            row_ids = (i * TR + k * SUB
                       + lax.broadcasted_iota(jnp.int32, (SUB, 1), 0))
            tri = col_ids > row_ids
            mdx = mxi - mxj
            mdy = myi - myj
            md2 = mdx * mdx + mdy * mdy
            prox = hi + hj + PROX
            mask = (~adj) & tri & (md2 < prox * prox)
            maskf = mask.astype(jnp.float32)

            b = dxi * dxj + dyi * dyj
            rx = sxi - sxj
            ry = syi - syj
            c = dxi * rx + dyi * ry
            f = dxj * rx + dyj * ry
            rdenom = 1.0 / jnp.maximum(ai * ej - b * b, 1e-12)
            s = jnp.clip((b * f - c * ej) * rdenom, 0.0, 1.0)
            t = jnp.clip((b * s + f) * rej, 0.0, 1.0)
            s = jnp.clip((b * t - c) * rai, 0.0, 1.0)
            ddx = rx + s * dxi - t * dxj
            ddy = ry + s * dyi - t * dyj
            sq = ddx * ddx + ddy * ddy
            dist = jnp.sqrt(jnp.maximum(sq, 1e-24))
            per = jnp.maximum(EPS - dist, 0.0) * maskf

            tp = tp + per
            tc = tc + maskf
        accs_ref[...] += tp
        accc_ref[...] += tc

    @pl.when((i == pl.num_programs(0) - 1) & (j == pl.num_programs(1) - 1))
    def _final():
        total = jnp.sum(accs_ref[...])
        cnt = jnp.sum(accc_ref[...])
        loss = jnp.where(cnt > 0.0, total / jnp.maximum(cnt, 1.0), 0.0)
        out_ref[...] = loss.reshape(1, 1)


def kernel(node_positions, adjacency, edge_index, weight):
    del adjacency, weight
    pos = node_positions.reshape(-1, 2)
    srcf = edge_index[0].astype(jnp.float32).reshape(E, 1)
    dstf = edge_index[1].astype(jnp.float32).reshape(E, 1)

    rows, cols = pl.pallas_call(
        _prep_body,
        grid=(E // PT,),
        in_specs=[
            pl.BlockSpec((E, 2), lambda i: (0, 0)),
            pl.BlockSpec((PT, 1), lambda i: (i, 0)),
            pl.BlockSpec((PT, 1), lambda i: (i, 0)),
        ],
        out_specs=[
            pl.BlockSpec((PT, NF), lambda i: (i, 0)),
            pl.BlockSpec((NF, PT), lambda i: (0, i)),
        ],
        out_shape=[
            jax.ShapeDtypeStruct((E, NF), jnp.float32),
            jax.ShapeDtypeStruct((NF, E), jnp.float32),
        ],
    )(pos, srcf, dstf)

    loss = pl.pallas_call(
        _pair_body,
        grid=(E // TR, E // TCOL),
        in_specs=[
            pl.BlockSpec((TR, NF), lambda i, j: (i, 0)),
            pl.BlockSpec((NF, TCOL), lambda i, j: (0, j)),
        ],
        out_specs=pl.BlockSpec((1, 1), lambda i, j: (0, 0)),
        out_shape=jax.ShapeDtypeStruct((1, 1), jnp.float32),
        scratch_shapes=[
            pltpu.VMEM((SUB, TCOL), jnp.float32),
            pltpu.VMEM((SUB, TCOL), jnp.float32),
        ],
    )(rows, cols)
    return loss.reshape(())


# speedup vs baseline: 3.6399x; 1.3835x over previous
import jax
import jax.numpy as jnp
from jax import lax
from jax.experimental import pallas as pl
from jax.experimental.pallas import tpu as pltpu

EPS = 0.001
PROX = 0.15

E = 2048
TR = 256
TCOL = 256
SUB = 32
PT = 256
NF = 16


def _prep(pos_ref, srcf_ref, dstf_ref, rows_s, cols_s):
    pos = pos_ref[...]
    for t in range(E // PT):
        sf = srcf_ref[t * PT:(t + 1) * PT, :]
        df = dstf_ref[t * PT:(t + 1) * PT, :]
        node_ids = lax.broadcasted_iota(jnp.int32, (PT, E), 1).astype(
            jnp.float32)
        oh_s = (sf == node_ids).astype(jnp.float32)
        oh_d = (df == node_ids).astype(jnp.float32)
        dn = (((1,), (0,)), ((), ()))
        ps = lax.dot_general(oh_s, pos, dn, preferred_element_type=jnp.float32)
        pd = lax.dot_general(oh_d, pos, dn, preferred_element_type=jnp.float32)
        sx = ps[:, 0:1]
        sy = ps[:, 1:2]
        ex = pd[:, 0:1]
        ey = pd[:, 1:2]
        dx = ex - sx
        dy = ey - sy
        len2 = dx * dx + dy * dy
        a = jnp.maximum(len2, 1e-12)
        ra = 1.0 / a
        mx = (sx + ex) * 0.5
        my = (sy + ey) * 0.5
        h = jnp.sqrt(len2) * 0.5
        z = jnp.zeros_like(sx)
        tile = jnp.concatenate(
            [sx, sy, dx, dy, a, ra, mx, my, h, sf, df, z, z, z, z, z], axis=1)
        rows_s[t * PT:(t + 1) * PT, :] = tile
        cols_s[:, t * PT:(t + 1) * PT] = tile.T


def _body(pos_ref, srcf_ref, dstf_ref, out_ref, rows_s, cols_s,
          accs_ref, accc_ref):
    i = pl.program_id(0)
    j = pl.program_id(1)

    @pl.when((i == 0) & (j == 0))
    def _init():
        _prep(pos_ref, srcf_ref, dstf_ref, rows_s, cols_s)
        accs_ref[...] = jnp.zeros((SUB, TCOL), jnp.float32)
        accc_ref[...] = jnp.zeros((SUB, TCOL), jnp.float32)

    @pl.when(j >= i)
    def _compute():
        joff = pl.multiple_of(j * TCOL, TCOL)
        cb = cols_s[:, pl.ds(joff, TCOL)]
        sxj = cb[0:1, :]
        syj = cb[1:2, :]
        dxj = cb[2:3, :]
        dyj = cb[3:4, :]
        ej = cb[4:5, :]
        rej = cb[5:6, :]
        mxj = cb[6:7, :]
        myj = cb[7:8, :]
        hj = cb[8:9, :]
        sfj = cb[9:10, :]
        dfj = cb[10:11, :]
        col_ids = j * TCOL + lax.broadcasted_iota(jnp.int32, (1, TCOL), 1)

        tp = jnp.zeros((SUB, TCOL), jnp.float32)
        tc = jnp.zeros((SUB, TCOL), jnp.float32)
        for k in range(TR // SUB):
            roff = pl.multiple_of(i * TR + k * SUB, SUB)
            rb = rows_s[pl.ds(roff, SUB), :]
            sxi = rb[:, 0:1]
            syi = rb[:, 1:2]
            dxi = rb[:, 2:3]
            dyi = rb[:, 3:4]
            ai = rb[:, 4:5]
            rai = rb[:, 5:6]
            mxi = rb[:, 6:7]
            myi = rb[:, 7:8]
            hi = rb[:, 8:9]
            sfi = rb[:, 9:10]
            dfi = rb[:, 10:11]

            adj = ((sfi == sfj) | (sfi == dfj) | (dfi == sfj) | (dfi == dfj))
            row_ids = (i * TR + k * SUB
                       + lax.broadcasted_iota(jnp.int32, (SUB, 1), 0))
            tri = col_ids > row_ids
            mdx = mxi - mxj
            mdy = myi - myj
            md2 = mdx * mdx + mdy * mdy
            prox = hi + hj + PROX
            mask = (~adj) & tri & (md2 < prox * prox)
            maskf = mask.astype(jnp.float32)

            b = dxi * dxj + dyi * dyj
            rx = sxi - sxj
            ry = syi - syj
            c = dxi * rx + dyi * ry
            f = dxj * rx + dyj * ry
            rdenom = 1.0 / jnp.maximum(ai * ej - b * b, 1e-12)
            s = jnp.clip((b * f - c * ej) * rdenom, 0.0, 1.0)
            t = jnp.clip((b * s + f) * rej, 0.0, 1.0)
            s = jnp.clip((b * t - c) * rai, 0.0, 1.0)
            ddx = rx + s * dxi - t * dxj
            ddy = ry + s * dyi - t * dyj
            sq = ddx * ddx + ddy * ddy
            dist = jnp.sqrt(jnp.maximum(sq, 1e-24))
            per = jnp.maximum(EPS - dist, 0.0) * maskf

            tp = tp + per
            tc = tc + maskf
        accs_ref[...] += tp
        accc_ref[...] += tc

    @pl.when((i == pl.num_programs(0) - 1) & (j == pl.num_programs(1) - 1))
    def _final():
        total = jnp.sum(accs_ref[...])
        cnt = jnp.sum(accc_ref[...])
        loss = jnp.where(cnt > 0.0, total / jnp.maximum(cnt, 1.0), 0.0)
        out_ref[...] = loss.reshape(1, 1)


def kernel(node_positions, adjacency, edge_index, weight):
    del adjacency, weight
    pos = node_positions.reshape(-1, 2)
    srcf = edge_index[0].astype(jnp.float32).reshape(E, 1)
    dstf = edge_index[1].astype(jnp.float32).reshape(E, 1)

    loss = pl.pallas_call(
        _body,
        grid=(E // TR, E // TCOL),
        in_specs=[
            pl.BlockSpec((E, 2), lambda i, j: (0, 0)),
            pl.BlockSpec((E, 1), lambda i, j: (0, 0)),
            pl.BlockSpec((E, 1), lambda i, j: (0, 0)),
        ],
        out_specs=pl.BlockSpec((1, 1), lambda i, j: (0, 0)),
        out_shape=jax.ShapeDtypeStruct((1, 1), jnp.float32),
        scratch_shapes=[
            pltpu.VMEM((E, NF), jnp.float32),
            pltpu.VMEM((NF, E), jnp.float32),
            pltpu.VMEM((SUB, TCOL), jnp.float32),
            pltpu.VMEM((SUB, TCOL), jnp.float32),
        ],
    )(pos, srcf, dstf)
    return loss.reshape(())
